# baseline (speedup 1.0000x reference)
import functools

import jax
import jax.numpy as jnp
from jax import lax
from jax.experimental import pallas as pl
from jax.experimental.pallas import tpu as pltpu
from jax.experimental.pallas import tpu_sc as plsc

_R = 200
_K = 45
_G = 200
_THRESH = 0.5
_LANES = 16


def _dense_body(conf_ref, obj_ref, istar_ref, p1_ref, cm_ref, ca_ref):
    x = conf_ref[...]
    root = x[:, :_R, :]
    m0 = jnp.max(root, axis=1, keepdims=True)
    ci = lax.broadcasted_iota(jnp.int32, root.shape, 1)
    istar_ref[...] = jnp.min(jnp.where(root >= m0, ci, _R), axis=1,
                             keepdims=True)
    p1_ref[...] = obj_ref[...] * m0
    m = conf_ref[:, _R:_R + _G * _K:_K, :]
    jm = jnp.zeros(m.shape, jnp.int32)
    for j in range(1, _K):
        v = conf_ref[:, _R + j:_R + _G * _K:_K, :]
        gt = v > m
        m = jnp.where(gt, v, m)
        jm = jnp.where(gt, j, jm)
    cm_ref[...] = m
    ca_ref[...] = jm


def _out_body(fn_ref, fp_ref, out_ref, *, bc, c_out):
    cb = pl.program_id(1)
    shape = out_ref.shape
    ci = lax.broadcasted_iota(jnp.int32, shape, 1) + cb * bc
    fn = fn_ref[...]
    fp = fp_ref[...]
    hit = jnp.logical_or(ci == fn, ci == c_out - 1)
    out_ref[...] = jnp.where(hit, fp, jnp.zeros(shape, jnp.float32))


def _make_sc_kernel(P, lpw, nc, ns, S, mesh):
    nb = lpw // _LANES

    def _sc_body(cm_t, ca_t, istar_h, p1_h, base_h, fn_h, fp_h,
                 istar_v, p1_v, base_v, idx_v, cm_v, ca_v, fn_v, fp_v, sem):
        wid = lax.axis_index("s") * nc + lax.axis_index("c")
        off = wid * lpw
        pltpu.sync_copy(istar_h.at[pl.ds(off, lpw)], istar_v)
        pltpu.sync_copy(p1_h.at[pl.ds(off, lpw)], p1_v)
        pltpu.sync_copy(base_h.at[pl.ds(off, lpw)], base_v)
        for b in range(nb):
            sl = pl.ds(b * _LANES, _LANES)
            idx_v[sl] = base_v[sl] + istar_v[sl] * S
        pltpu.async_copy(cm_t.at[idx_v], cm_v, sem).wait()
        pltpu.async_copy(ca_t.at[idx_v], ca_v, sem).wait()
        for b in range(nb):
            sl = pl.ds(b * _LANES, _LANES)
            ist = istar_v[sl]
            p1b = p1_v[sl]
            p2 = p1b * cm_v[sl]
            take = jnp.logical_and(p1b > _THRESH, p2 > _THRESH)
            fn_v[sl] = jnp.where(take, (_R + ist * _K) + ca_v[sl], ist)
            fp_v[sl] = jnp.where(take, p2, p1b)
        pltpu.sync_copy(fn_v, fn_h.at[pl.ds(off, lpw)])
        pltpu.sync_copy(fp_v, fp_h.at[pl.ds(off, lpw)])

    return pl.kernel(
        _sc_body,
        out_type=[jax.ShapeDtypeStruct((P,), jnp.int32),
                  jax.ShapeDtypeStruct((P,), jnp.float32)],
        mesh=mesh,
        scratch_types=[
            pltpu.VMEM((lpw,), jnp.int32),
            pltpu.VMEM((lpw,), jnp.float32),
            pltpu.VMEM((lpw,), jnp.int32),
            pltpu.VMEM((lpw,), jnp.int32),
            pltpu.VMEM((lpw,), jnp.float32),
            pltpu.VMEM((lpw,), jnp.int32),
            pltpu.VMEM((lpw,), jnp.int32),
            pltpu.VMEM((lpw,), jnp.float32),
            pltpu.SemaphoreType.DMA,
        ],
    )


def kernel(conf, obj):
    N, C, S = conf.shape
    NS = N * S

    bS = 128
    nsb = -(-S // bS)
    i_star, p1, cm, ca = pl.pallas_call(
        _dense_body,
        grid=(N, nsb),
        in_specs=[
            pl.BlockSpec((1, C, bS), lambda n, sb: (n, 0, sb)),
            pl.BlockSpec((1, 1, bS), lambda n, sb: (n, 0, sb)),
        ],
        out_specs=[
            pl.BlockSpec((1, 1, bS), lambda n, sb: (n, 0, sb)),
            pl.BlockSpec((1, 1, bS), lambda n, sb: (n, 0, sb)),
            pl.BlockSpec((1, _G, bS), lambda n, sb: (n, 0, sb)),
            pl.BlockSpec((1, _G, bS), lambda n, sb: (n, 0, sb)),
        ],
        out_shape=[jax.ShapeDtypeStruct((N, 1, S), jnp.int32),
                   jax.ShapeDtypeStruct((N, 1, S), jnp.float32),
                   jax.ShapeDtypeStruct((N, _G, S), jnp.float32),
                   jax.ShapeDtypeStruct((N, _G, S), jnp.int32)],
        compiler_params=pltpu.CompilerParams(
            dimension_semantics=("parallel", "parallel")),
    )(conf, obj.reshape(N, 1, S))

    mesh = plsc.VectorSubcoreMesh(core_axis_name="c", subcore_axis_name="s")
    nc, ns = mesh.num_cores, mesh.num_subcores
    nw = nc * ns
    lpw = -(-NS // nw)
    lpw = -(-lpw // _LANES) * _LANES
    P = nw * lpw

    loc = jnp.minimum(jnp.arange(P, dtype=jnp.int32), NS - 1)
    n_ = loc // S
    s_ = loc - n_ * S
    base = n_ * (_G * S) + s_

    pad = P - NS
    istar_p = jnp.concatenate([i_star.reshape(-1),
                               jnp.zeros((pad,), jnp.int32)])
    p1_p = jnp.concatenate([p1.reshape(-1), jnp.zeros((pad,), jnp.float32)])
    cm_t = cm.reshape(N * _G * S)
    ca_t = ca.reshape(N * _G * S)

    sc_fn = _make_sc_kernel(P, lpw, nc, ns, S, mesh)
    fn_p, fp_p = sc_fn(cm_t, ca_t, istar_p, p1_p, base)
    fn = fn_p[:NS].reshape(N, 1, S)
    fp = fp_p[:NS].reshape(N, 1, S)

    c_out = C + 1
    bc = 2304
    n_cb = -(-c_out // bc)
    out = pl.pallas_call(
        functools.partial(_out_body, bc=bc, c_out=c_out),
        grid=(N, n_cb),
        in_specs=[
            pl.BlockSpec((1, 1, S), lambda n, cb: (n, 0, 0)),
            pl.BlockSpec((1, 1, S), lambda n, cb: (n, 0, 0)),
        ],
        out_specs=pl.BlockSpec((1, bc, S), lambda n, cb: (n, cb, 0)),
        out_shape=jax.ShapeDtypeStruct((N, c_out, S), jnp.float32),
        compiler_params=pltpu.CompilerParams(
            dimension_semantics=("parallel", "parallel")),
    )(fn, fp)
    return out


# speedup vs baseline: 1.0107x; 1.0107x over previous
import functools

import jax
import jax.numpy as jnp
from jax import lax
from jax.experimental import pallas as pl
from jax.experimental.pallas import tpu as pltpu
from jax.experimental.pallas import tpu_sc as plsc

_R = 200
_K = 45
_G = 200
_THRESH = 0.5
_LANES = 16


def _dense_body(conf_ref, obj_ref, istar_ref, p1_ref, cm_ref, ca_ref):
    x = conf_ref[...]
    root = x[:, :_R, :]
    m0 = jnp.max(root, axis=1, keepdims=True)
    ci = lax.broadcasted_iota(jnp.int32, root.shape, 1)
    istar_ref[...] = jnp.min(jnp.where(root >= m0, ci, _R), axis=1,
                             keepdims=True)
    p1_ref[...] = obj_ref[...] * m0
    m = conf_ref[:, _R:_R + _G * _K:_K, :]
    jm = jnp.zeros(m.shape, jnp.int32)
    for j in range(1, _K):
        v = conf_ref[:, _R + j:_R + _G * _K:_K, :]
        gt = v > m
        m = jnp.where(gt, v, m)
        jm = jnp.where(gt, j, jm)
    cm_ref[...] = m
    ca_ref[...] = jm


def _out_body(fn_ref, fp_ref, out_ref, *, bc, c_out, S, SP):
    n = pl.program_id(0)
    cb = pl.program_id(1)
    shape = out_ref.shape
    ci = lax.broadcasted_iota(jnp.int32, shape, 1) + cb * bc
    fn = fn_ref[pl.ds(n * SP, SP)][:S][None, None, :]
    fp = fp_ref[pl.ds(n * SP, SP)][:S][None, None, :]
    hit = jnp.logical_or(ci == fn, ci == c_out - 1)
    out_ref[...] = jnp.where(hit, fp, jnp.zeros(shape, jnp.float32))


def _make_sc_kernel(P, lpw, nc, ns, S, mesh):
    nb = lpw // _LANES

    def _sc_body(cm_t, ca_t, istar_h, p1_h, loc_h, base_h, fn_h, fp_h,
                 istar_v, p1_v, loc_v, base_v, idx_v, cm_v, ca_v, fn_v, fp_v,
                 sem):
        wid = lax.axis_index("s") * nc + lax.axis_index("c")
        off = jnp.minimum(wid * lpw, P - lpw)
        pltpu.sync_copy(loc_h.at[pl.ds(off, lpw)], loc_v)
        pltpu.sync_copy(base_h.at[pl.ds(off, lpw)], base_v)
        pltpu.async_copy(istar_h.at[loc_v], istar_v, sem).wait()
        pltpu.async_copy(p1_h.at[loc_v], p1_v, sem).wait()
        for b in range(nb):
            sl = pl.ds(b * _LANES, _LANES)
            idx_v[sl] = base_v[sl] + istar_v[sl] * S
        pltpu.async_copy(cm_t.at[idx_v], cm_v, sem).wait()
        pltpu.async_copy(ca_t.at[idx_v], ca_v, sem).wait()
        for b in range(nb):
            sl = pl.ds(b * _LANES, _LANES)
            ist = istar_v[sl]
            p1b = p1_v[sl]
            p2 = p1b * cm_v[sl]
            take = jnp.logical_and(p1b > _THRESH, p2 > _THRESH)
            fn_v[sl] = jnp.where(take, (_R + ist * _K) + ca_v[sl], ist)
            fp_v[sl] = jnp.where(take, p2, p1b)
        pltpu.sync_copy(fn_v, fn_h.at[pl.ds(off, lpw)])
        pltpu.sync_copy(fp_v, fp_h.at[pl.ds(off, lpw)])

    return pl.kernel(
        _sc_body,
        out_type=[jax.ShapeDtypeStruct((P,), jnp.int32),
                  jax.ShapeDtypeStruct((P,), jnp.float32)],
        mesh=mesh,
        scratch_types=[
            pltpu.VMEM((lpw,), jnp.int32),
            pltpu.VMEM((lpw,), jnp.float32),
            pltpu.VMEM((lpw,), jnp.int32),
            pltpu.VMEM((lpw,), jnp.int32),
            pltpu.VMEM((lpw,), jnp.int32),
            pltpu.VMEM((lpw,), jnp.float32),
            pltpu.VMEM((lpw,), jnp.int32),
            pltpu.VMEM((lpw,), jnp.int32),
            pltpu.VMEM((lpw,), jnp.float32),
            pltpu.SemaphoreType.DMA,
        ],
    )


def kernel(conf, obj):
    N, C, S = conf.shape
    NS = N * S

    bS = 128
    nsb = -(-S // bS)
    i_star, p1, cm, ca = pl.pallas_call(
        _dense_body,
        grid=(N, nsb),
        in_specs=[
            pl.BlockSpec((1, C, bS), lambda n, sb: (n, 0, sb)),
            pl.BlockSpec((1, 1, bS), lambda n, sb: (n, 0, sb)),
        ],
        out_specs=[
            pl.BlockSpec((1, 1, bS), lambda n, sb: (n, 0, sb)),
            pl.BlockSpec((1, 1, bS), lambda n, sb: (n, 0, sb)),
            pl.BlockSpec((1, _G, bS), lambda n, sb: (n, 0, sb)),
            pl.BlockSpec((1, _G, bS), lambda n, sb: (n, 0, sb)),
        ],
        out_shape=[jax.ShapeDtypeStruct((N, 1, S), jnp.int32),
                   jax.ShapeDtypeStruct((N, 1, S), jnp.float32),
                   jax.ShapeDtypeStruct((N, _G, S), jnp.float32),
                   jax.ShapeDtypeStruct((N, _G, S), jnp.int32)],
        compiler_params=pltpu.CompilerParams(
            dimension_semantics=("parallel", "parallel")),
    )(conf, obj.reshape(N, 1, S))

    mesh = plsc.VectorSubcoreMesh(core_axis_name="c", subcore_axis_name="s")
    nc, ns = mesh.num_cores, mesh.num_subcores
    nw = nc * ns
    SP = -(-S // 128) * 128
    P = N * SP
    lpw = -(-P // nw)
    lpw = -(-lpw // _LANES) * _LANES

    i = jnp.arange(P, dtype=jnp.int32)
    n_ = i // SP
    s_ = jnp.minimum(i - n_ * SP, S - 1)
    loc = n_ * S + s_
    base = n_ * (_G * S) + s_

    cm_t = cm.reshape(N * _G * S)
    ca_t = ca.reshape(N * _G * S)

    sc_fn = _make_sc_kernel(P, lpw, nc, ns, S, mesh)
    fn_p, fp_p = sc_fn(cm_t, ca_t, i_star.reshape(-1), p1.reshape(-1),
                       loc, base)

    c_out = C + 1
    bc = 2304
    n_cb = -(-c_out // bc)
    out = pl.pallas_call(
        functools.partial(_out_body, bc=bc, c_out=c_out, S=S, SP=SP),
        grid=(N, n_cb),
        in_specs=[
            pl.BlockSpec((P,), lambda n, cb: (0,)),
            pl.BlockSpec((P,), lambda n, cb: (0,)),
        ],
        out_specs=pl.BlockSpec((1, bc, S), lambda n, cb: (n, cb, 0)),
        out_shape=jax.ShapeDtypeStruct((N, c_out, S), jnp.float32),
        compiler_params=pltpu.CompilerParams(
            dimension_semantics=("parallel", "parallel")),
    )(fn_p, fp_p)
    return out


# speedup vs baseline: 1.0166x; 1.0058x over previous
import functools

import jax
import jax.numpy as jnp
from jax import lax
from jax.experimental import pallas as pl
from jax.experimental.pallas import tpu as pltpu
from jax.experimental.pallas import tpu_sc as plsc

_R = 200
_K = 45
_G = 200
_THRESH = 0.5
_LANES = 16


def _dense_body(conf_ref, obj_ref, istar_ref, p1_ref, cm_ref):
    x = conf_ref[...]
    root = x[:, :_R, :]
    m0 = jnp.max(root, axis=1, keepdims=True)
    ci = lax.broadcasted_iota(jnp.int32, root.shape, 1)
    istar_ref[...] = jnp.min(jnp.where(root >= m0, ci, _R), axis=1,
                             keepdims=True)
    p1_ref[...] = obj_ref[...] * m0
    m = conf_ref[:, _R:_R + _G * _K:_K, :]
    jm = jnp.zeros(m.shape, jnp.float32)
    for j in range(1, _K):
        v = conf_ref[:, _R + j:_R + _G * _K:_K, :]
        gt = v > m
        m = jnp.where(gt, v, m)
        jm = jnp.where(gt, jnp.float32(j), jm)
    cm_ref[:, :_G, :] = m
    cm_ref[:, _G:, :] = jm


def _out_body(fn_ref, fp_ref, out_ref, *, bc, c_out, S, SP):
    n = pl.program_id(0)
    cb = pl.program_id(1)
    shape = out_ref.shape
    ci = lax.broadcasted_iota(jnp.int32, shape, 1) + cb * bc
    fn = fn_ref[pl.ds(n * SP, SP)][:S][None, None, :]
    fp = fp_ref[pl.ds(n * SP, SP)][:S][None, None, :]
    cif = ci.astype(jnp.float32)
    hit = jnp.logical_or(cif == fn, ci == c_out - 1)
    out_ref[...] = jnp.where(hit, fp, jnp.zeros(shape, jnp.float32))


def _make_sc_kernel(P, lpw, nc, ns, S, mesh):
    nb = lpw // _LANES

    def _sc_body(cm_t, istar_h, p1_h, loc_h, base_h, fn_h, fp_h,
                 istar_v, p1_v, loc_v, base_v, idx_v, vals_v, fn_v, fp_v,
                 sem):
        wid = lax.axis_index("s") * nc + lax.axis_index("c")
        off = jnp.minimum(wid * lpw, P - lpw)
        pltpu.sync_copy(loc_h.at[pl.ds(off, lpw)], loc_v)
        pltpu.sync_copy(base_h.at[pl.ds(off, lpw)], base_v)
        pltpu.async_copy(istar_h.at[loc_v], istar_v, sem).wait()
        pltpu.async_copy(p1_h.at[loc_v], p1_v, sem).wait()
        for b in range(nb):
            sl = pl.ds(b * _LANES, _LANES)
            cmi = base_v[sl] + istar_v[sl] * S
            idx_v[sl] = cmi
            idx_v[pl.ds(lpw + b * _LANES, _LANES)] = cmi + _G * S
        pltpu.async_copy(cm_t.at[idx_v], vals_v, sem).wait()
        for b in range(nb):
            sl = pl.ds(b * _LANES, _LANES)
            istf = istar_v[sl].astype(jnp.float32)
            p1b = p1_v[sl]
            p2 = p1b * vals_v[sl]
            jmf = vals_v[pl.ds(lpw + b * _LANES, _LANES)]
            take = jnp.logical_and(p1b > _THRESH, p2 > _THRESH)
            fn_v[sl] = jnp.where(take, (_R + istf * _K) + jmf, istf)
            fp_v[sl] = jnp.where(take, p2, p1b)
        pltpu.sync_copy(fn_v, fn_h.at[pl.ds(off, lpw)])
        pltpu.sync_copy(fp_v, fp_h.at[pl.ds(off, lpw)])

    return pl.kernel(
        _sc_body,
        out_type=[jax.ShapeDtypeStruct((P,), jnp.float32),
                  jax.ShapeDtypeStruct((P,), jnp.float32)],
        mesh=mesh,
        scratch_types=[
            pltpu.VMEM((lpw,), jnp.int32),
            pltpu.VMEM((lpw,), jnp.float32),
            pltpu.VMEM((lpw,), jnp.int32),
            pltpu.VMEM((lpw,), jnp.int32),
            pltpu.VMEM((2 * lpw,), jnp.int32),
            pltpu.VMEM((2 * lpw,), jnp.float32),
            pltpu.VMEM((lpw,), jnp.float32),
            pltpu.VMEM((lpw,), jnp.float32),
            pltpu.SemaphoreType.DMA,
        ],
    )


def kernel(conf, obj):
    N, C, S = conf.shape
    NS = N * S

    bS = 128
    nsb = -(-S // bS)
    i_star, p1, cm = pl.pallas_call(
        _dense_body,
        grid=(N, nsb),
        in_specs=[
            pl.BlockSpec((1, C, bS), lambda n, sb: (n, 0, sb)),
            pl.BlockSpec((1, 1, bS), lambda n, sb: (n, 0, sb)),
        ],
        out_specs=[
            pl.BlockSpec((1, 1, bS), lambda n, sb: (n, 0, sb)),
            pl.BlockSpec((1, 1, bS), lambda n, sb: (n, 0, sb)),
            pl.BlockSpec((1, 2 * _G, bS), lambda n, sb: (n, 0, sb)),
        ],
        out_shape=[jax.ShapeDtypeStruct((N, 1, S), jnp.int32),
                   jax.ShapeDtypeStruct((N, 1, S), jnp.float32),
                   jax.ShapeDtypeStruct((N, 2 * _G, S), jnp.float32)],
        compiler_params=pltpu.CompilerParams(
            dimension_semantics=("parallel", "parallel")),
    )(conf, obj.reshape(N, 1, S))

    mesh = plsc.VectorSubcoreMesh(core_axis_name="c", subcore_axis_name="s")
    nc, ns = mesh.num_cores, mesh.num_subcores
    nw = nc * ns
    SP = -(-S // 128) * 128
    P = N * SP
    lpw = -(-P // nw)
    lpw = -(-lpw // _LANES) * _LANES

    i = jnp.arange(P, dtype=jnp.int32)
    n_ = i // SP
    s_ = jnp.minimum(i - n_ * SP, S - 1)
    loc = n_ * S + s_
    base = n_ * (2 * _G * S) + s_

    cm_t = cm.reshape(N * 2 * _G * S)

    sc_fn = _make_sc_kernel(P, lpw, nc, ns, S, mesh)
    fn_p, fp_p = sc_fn(cm_t, i_star.reshape(-1), p1.reshape(-1),
                       loc, base)

    c_out = C + 1
    bc = 2304
    n_cb = -(-c_out // bc)
    out = pl.pallas_call(
        functools.partial(_out_body, bc=bc, c_out=c_out, S=S, SP=SP),
        grid=(N, n_cb),
        in_specs=[
            pl.BlockSpec((P,), lambda n, cb: (0,)),
            pl.BlockSpec((P,), lambda n, cb: (0,)),
        ],
        out_specs=pl.BlockSpec((1, bc, S), lambda n, cb: (n, cb, 0)),
        out_shape=jax.ShapeDtypeStruct((N, c_out, S), jnp.float32),
        compiler_params=pltpu.CompilerParams(
            dimension_semantics=("parallel", "parallel")),
    )(fn_p, fp_p)
    return out


# speedup vs baseline: 1.0347x; 1.0178x over previous
import functools

import jax
import jax.numpy as jnp
from jax import lax
from jax.experimental import pallas as pl
from jax.experimental.pallas import tpu as pltpu
from jax.experimental.pallas import tpu_sc as plsc

_R = 200
_K = 45
_G = 200
_THRESH = 0.5
_LANES = 16


def _dense_body(conf_ref, obj_ref, istar_ref, p1_ref, cm_ref):
    x = conf_ref[...]
    root = x[:, :_R, :]
    m0 = jnp.max(root, axis=1, keepdims=True)
    ci = lax.broadcasted_iota(jnp.int32, root.shape, 1)
    istar_ref[...] = jnp.min(jnp.where(root >= m0, ci, _R), axis=1,
                             keepdims=True)
    p1_ref[...] = obj_ref[...] * m0
    m = conf_ref[:, _R:_R + _G * _K:_K, :]
    jm = jnp.zeros(m.shape, jnp.float32)
    for j in range(1, _K):
        v = conf_ref[:, _R + j:_R + _G * _K:_K, :]
        gt = v > m
        m = jnp.where(gt, v, m)
        jm = jnp.where(gt, jnp.float32(j), jm)
    cm_ref[0, 0, :_G, :] = m[0]
    cm_ref[0, 0, _G:, :] = jm[0]


def _out_body(fn_ref, fp_ref, out_ref, *, bc, c_out, S, SP):
    n = pl.program_id(0)
    cb = pl.program_id(1)
    shape = out_ref.shape
    ci = lax.broadcasted_iota(jnp.int32, shape, 1) + cb * bc
    fn = fn_ref[pl.ds(n * SP, SP)][:S][None, None, :]
    fp = fp_ref[pl.ds(n * SP, SP)][:S][None, None, :]
    cif = ci.astype(jnp.float32)
    hit = jnp.logical_or(cif == fn, ci == c_out - 1)
    out_ref[...] = jnp.where(hit, fp, jnp.zeros(shape, jnp.float32))


def _make_sc_kernel(P, lpw, nc, ns, S, mesh):
    nb = lpw // _LANES

    def _sc_body(cm_t, istar_h, p1_h, loc_h, base_h, fn_h, fp_h,
                 istar_v, p1_v, loc_v, base_v, idx_v, vals_v, fn_v, fp_v,
                 sem):
        wid = lax.axis_index("s") * nc + lax.axis_index("c")
        off = jnp.minimum(wid * lpw, P - lpw)
        pltpu.sync_copy(loc_h.at[pl.ds(off, lpw)], loc_v)
        pltpu.sync_copy(base_h.at[pl.ds(off, lpw)], base_v)
        pltpu.async_copy(istar_h.at[loc_v], istar_v, sem).wait()
        pltpu.async_copy(p1_h.at[loc_v], p1_v, sem).wait()
        for b in range(nb):
            sl = pl.ds(b * _LANES, _LANES)
            cmi = base_v[sl] + istar_v[sl] * 128
            idx_v[sl] = cmi
            idx_v[pl.ds(lpw + b * _LANES, _LANES)] = cmi + _G * 128
        pltpu.async_copy(cm_t.at[idx_v], vals_v, sem).wait()
        for b in range(nb):
            sl = pl.ds(b * _LANES, _LANES)
            istf = istar_v[sl].astype(jnp.float32)
            p1b = p1_v[sl]
            p2 = p1b * vals_v[sl]
            jmf = vals_v[pl.ds(lpw + b * _LANES, _LANES)]
            take = jnp.logical_and(p1b > _THRESH, p2 > _THRESH)
            fn_v[sl] = jnp.where(take, (_R + istf * _K) + jmf, istf)
            fp_v[sl] = jnp.where(take, p2, p1b)
        pltpu.sync_copy(fn_v, fn_h.at[pl.ds(off, lpw)])
        pltpu.sync_copy(fp_v, fp_h.at[pl.ds(off, lpw)])

    return pl.kernel(
        _sc_body,
        out_type=[jax.ShapeDtypeStruct((P,), jnp.float32),
                  jax.ShapeDtypeStruct((P,), jnp.float32)],
        mesh=mesh,
        scratch_types=[
            pltpu.VMEM((lpw,), jnp.int32),
            pltpu.VMEM((lpw,), jnp.float32),
            pltpu.VMEM((lpw,), jnp.int32),
            pltpu.VMEM((lpw,), jnp.int32),
            pltpu.VMEM((2 * lpw,), jnp.int32),
            pltpu.VMEM((2 * lpw,), jnp.float32),
            pltpu.VMEM((lpw,), jnp.float32),
            pltpu.VMEM((lpw,), jnp.float32),
            pltpu.SemaphoreType.DMA,
        ],
    )


def kernel(conf, obj):
    N, C, S = conf.shape
    NS = N * S

    bS = 128
    nsb = -(-S // bS)
    i_star, p1, cm = pl.pallas_call(
        _dense_body,
        grid=(N, nsb),
        in_specs=[
            pl.BlockSpec((1, C, bS), lambda n, sb: (n, 0, sb)),
            pl.BlockSpec((1, 1, bS), lambda n, sb: (n, 0, sb)),
        ],
        out_specs=[
            pl.BlockSpec((1, 1, bS), lambda n, sb: (n, 0, sb)),
            pl.BlockSpec((1, 1, bS), lambda n, sb: (n, 0, sb)),
            pl.BlockSpec((1, 1, 2 * _G, bS), lambda n, sb: (n, sb, 0, 0)),
        ],
        out_shape=[jax.ShapeDtypeStruct((N, 1, S), jnp.int32),
                   jax.ShapeDtypeStruct((N, 1, S), jnp.float32),
                   jax.ShapeDtypeStruct((N, nsb, 2 * _G, bS), jnp.float32)],
        compiler_params=pltpu.CompilerParams(
            dimension_semantics=("parallel", "parallel")),
    )(conf, obj.reshape(N, 1, S))

    mesh = plsc.VectorSubcoreMesh(core_axis_name="c", subcore_axis_name="s")
    nc, ns = mesh.num_cores, mesh.num_subcores
    nw = nc * ns
    SP = -(-S // 128) * 128
    P = N * SP
    lpw = -(-P // nw)
    lpw = -(-lpw // _LANES) * _LANES

    i = jnp.arange(P, dtype=jnp.int32)
    n_ = i // SP
    s_ = jnp.minimum(i - n_ * SP, S - 1)
    loc = n_ * S + s_
    nsb = -(-S // 128)
    base = ((n_ * nsb + s_ // 128) * (2 * _G)) * 128 + s_ % 128

    cm_t = cm.reshape(-1)

    sc_fn = _make_sc_kernel(P, lpw, nc, ns, S, mesh)
    fn_p, fp_p = sc_fn(cm_t, i_star.reshape(-1), p1.reshape(-1),
                       loc, base)

    c_out = C + 1
    bc = 2304
    n_cb = -(-c_out // bc)
    out = pl.pallas_call(
        functools.partial(_out_body, bc=bc, c_out=c_out, S=S, SP=SP),
        grid=(N, n_cb),
        in_specs=[
            pl.BlockSpec((P,), lambda n, cb: (0,)),
            pl.BlockSpec((P,), lambda n, cb: (0,)),
        ],
        out_specs=pl.BlockSpec((1, bc, S), lambda n, cb: (n, cb, 0)),
        out_shape=jax.ShapeDtypeStruct((N, c_out, S), jnp.float32),
        compiler_params=pltpu.CompilerParams(
            dimension_semantics=("parallel", "parallel")),
    )(fn_p, fp_p)
    return out


# speedup vs baseline: 1.0557x; 1.0203x over previous
import functools

import jax
import jax.numpy as jnp
from jax import lax
from jax.experimental import pallas as pl
from jax.experimental.pallas import tpu as pltpu
from jax.experimental.pallas import tpu_sc as plsc

_R = 200
_K = 45
_G = 200
_THRESH = 0.5
_LANES = 16


def _dense_body(conf_ref, obj_ref, istar_ref, p1_ref, cm_ref):
    x = conf_ref[...]
    root = x[:, :_R, :]
    m0 = jnp.max(root, axis=1, keepdims=True)
    ci = lax.broadcasted_iota(jnp.int32, root.shape, 1)
    istar_ref[0, 0, 0, :] = jnp.min(jnp.where(root >= m0, ci, _R),
                                    axis=1)[0, :]
    p1_ref[0, 0, 0, :] = (obj_ref[...] * m0)[0, 0, :]
    m = conf_ref[:, _R:_R + _G * _K:_K, :]
    jm = jnp.zeros(m.shape, jnp.float32)
    for j in range(1, _K):
        v = conf_ref[:, _R + j:_R + _G * _K:_K, :]
        gt = v > m
        m = jnp.where(gt, v, m)
        jm = jnp.where(gt, jnp.float32(j), jm)
    cm_ref[0, 0, :_G, :] = m[0]
    cm_ref[0, 0, _G:, :] = jm[0]


def _out_body(fn_ref, fp_ref, out_ref, *, bc, c_out, S, SP):
    n = pl.program_id(0)
    cb = pl.program_id(1)
    shape = out_ref.shape
    ci = lax.broadcasted_iota(jnp.int32, shape, 1) + cb * bc
    fn = fn_ref[pl.ds(n * SP, SP)][:S][None, None, :]
    fp = fp_ref[pl.ds(n * SP, SP)][:S][None, None, :]
    cif = ci.astype(jnp.float32)
    hit = jnp.logical_or(cif == fn, ci == c_out - 1)
    out_ref[...] = jnp.where(hit, fp, jnp.zeros(shape, jnp.float32))


def _make_sc_kernel(P, lpw, nc, ns, S, mesh):
    nb = lpw // _LANES

    def _sc_body(cm_t, istar_h, p1_h, base_h, fn_h, fp_h,
                 istar_v, p1_v, base_v, idx_v, vals_v, fn_v, fp_v,
                 sem):
        wid = lax.axis_index("s") * nc + lax.axis_index("c")
        off = jnp.minimum(wid * lpw, P - lpw)
        pltpu.sync_copy(istar_h.at[pl.ds(off, lpw)], istar_v)
        pltpu.sync_copy(p1_h.at[pl.ds(off, lpw)], p1_v)
        pltpu.sync_copy(base_h.at[pl.ds(off, lpw)], base_v)
        for b in range(nb):
            sl = pl.ds(b * _LANES, _LANES)
            ist = jnp.clip(istar_v[sl], 0, _G - 1)
            cmi = base_v[sl] + ist * 128
            idx_v[sl] = cmi
            idx_v[pl.ds(lpw + b * _LANES, _LANES)] = cmi + _G * 128
        pltpu.async_copy(cm_t.at[idx_v], vals_v, sem).wait()
        for b in range(nb):
            sl = pl.ds(b * _LANES, _LANES)
            istf = istar_v[sl].astype(jnp.float32)
            p1b = p1_v[sl]
            p2 = p1b * vals_v[sl]
            jmf = vals_v[pl.ds(lpw + b * _LANES, _LANES)]
            take = jnp.logical_and(p1b > _THRESH, p2 > _THRESH)
            fn_v[sl] = jnp.where(take, (_R + istf * _K) + jmf, istf)
            fp_v[sl] = jnp.where(take, p2, p1b)
        pltpu.sync_copy(fn_v, fn_h.at[pl.ds(off, lpw)])
        pltpu.sync_copy(fp_v, fp_h.at[pl.ds(off, lpw)])

    return pl.kernel(
        _sc_body,
        out_type=[jax.ShapeDtypeStruct((P,), jnp.float32),
                  jax.ShapeDtypeStruct((P,), jnp.float32)],
        mesh=mesh,
        scratch_types=[
            pltpu.VMEM((lpw,), jnp.int32),
            pltpu.VMEM((lpw,), jnp.float32),
            pltpu.VMEM((lpw,), jnp.int32),
            pltpu.VMEM((2 * lpw,), jnp.int32),
            pltpu.VMEM((2 * lpw,), jnp.float32),
            pltpu.VMEM((lpw,), jnp.float32),
            pltpu.VMEM((lpw,), jnp.float32),
            pltpu.SemaphoreType.DMA,
        ],
    )


def kernel(conf, obj):
    N, C, S = conf.shape
    NS = N * S

    bS = 128
    nsb = -(-S // bS)
    i_star, p1, cm = pl.pallas_call(
        _dense_body,
        grid=(N, nsb),
        in_specs=[
            pl.BlockSpec((1, C, bS), lambda n, sb: (n, 0, sb)),
            pl.BlockSpec((1, 1, bS), lambda n, sb: (n, 0, sb)),
        ],
        out_specs=[
            pl.BlockSpec((1, 1, 1, bS), lambda n, sb: (n, sb, 0, 0)),
            pl.BlockSpec((1, 1, 1, bS), lambda n, sb: (n, sb, 0, 0)),
            pl.BlockSpec((1, 1, 2 * _G, bS), lambda n, sb: (n, sb, 0, 0)),
        ],
        out_shape=[jax.ShapeDtypeStruct((N, nsb, 1, bS), jnp.int32),
                   jax.ShapeDtypeStruct((N, nsb, 1, bS), jnp.float32),
                   jax.ShapeDtypeStruct((N, nsb, 2 * _G, bS), jnp.float32)],
        compiler_params=pltpu.CompilerParams(
            dimension_semantics=("parallel", "parallel")),
    )(conf, obj.reshape(N, 1, S))

    mesh = plsc.VectorSubcoreMesh(core_axis_name="c", subcore_axis_name="s")
    nc, ns = mesh.num_cores, mesh.num_subcores
    nw = nc * ns
    SP = -(-S // 128) * 128
    P = N * SP
    lpw = -(-P // nw)
    lpw = -(-lpw // _LANES) * _LANES

    i = jnp.arange(P, dtype=jnp.int32)
    n_ = i // SP
    s_ = jnp.minimum(i - n_ * SP, S - 1)
    base = ((n_ * nsb + s_ // 128) * (2 * _G)) * 128 + s_ % 128

    cm_t = cm.reshape(-1)

    sc_fn = _make_sc_kernel(P, lpw, nc, ns, S, mesh)
    fn_p, fp_p = sc_fn(cm_t, i_star.reshape(-1), p1.reshape(-1), base)

    c_out = C + 1
    bc = 2304
    n_cb = -(-c_out // bc)
    out = pl.pallas_call(
        functools.partial(_out_body, bc=bc, c_out=c_out, S=S, SP=SP),
        grid=(N, n_cb),
        in_specs=[
            pl.BlockSpec((P,), lambda n, cb: (0,)),
            pl.BlockSpec((P,), lambda n, cb: (0,)),
        ],
        out_specs=pl.BlockSpec((1, bc, S), lambda n, cb: (n, cb, 0)),
        out_shape=jax.ShapeDtypeStruct((N, c_out, S), jnp.float32),
        compiler_params=pltpu.CompilerParams(
            dimension_semantics=("parallel", "parallel")),
    )(fn_p, fp_p)
    return out
